# Initial kernel scaffold; baseline (speedup 1.0000x reference)
import jax
import jax.numpy as jnp
from jax.experimental import pallas as pl


def kernel(x, edge_index, W1, b1, W2, b2):
    raise NotImplementedError("write your pallas kernel here")


# speedup vs baseline: 4.8092x; 4.8092x over previous
import functools

import jax
import jax.numpy as jnp
from jax import lax
from jax.experimental import pallas as pl
from jax.experimental.pallas import tpu as pltpu
from jax.experimental.pallas import tpu_sc as plsc

EPS = 1e-7
ARTANH_LIM = 1.0 - 1e-7

N, D = 10000, 128
E = 320000
NC, NS = 2, 16
NW = NC * NS
EW = E // NW
CHUNK = 80
NCHUNK = EW // CHUNK
NP = 10240
ROWS_PER_SUB = NP // NS


def _tc1_body(x_ref, w1_ref, b1_ref, out_ref):
    xb = x_ref[...]
    nrm = jnp.maximum(jnp.sqrt(jnp.sum(xb * xb, axis=1, keepdims=True)), EPS)
    z = jnp.minimum(nrm, ARTANH_LIM)
    scale = 0.5 * jnp.log((1.0 + z) / (1.0 - z)) / nrm
    t = xb * scale
    out_ref[...] = lax.dot_general(
        t, w1_ref[...], (((1,), (1,)), ((), ())),
        preferred_element_type=jnp.float32) + b1_ref[...]


def _tc2_body(acc_ref, deg_ref, w2_ref, b2_ref, out_ref):
    s = acc_ref[0, :N] + acc_ref[1, :N]
    deg = deg_ref[0, :N] + deg_ref[1, :N]
    agg = s / jnp.maximum(deg, 1.0)
    h = lax.dot_general(
        agg, w2_ref[...], (((1,), (1,)), ((), ())),
        preferred_element_type=jnp.float32) + b2_ref[...]
    nrm = jnp.maximum(jnp.sqrt(jnp.sum(h * h, axis=1, keepdims=True)), EPS)
    out_ref[...] = jnp.maximum(h * (jnp.tanh(nrm) / nrm), 0.0)


def _segment_sum_sc_body(t1, src, dst, acc_out, deg_out,
                         acc_sp, srcb, dstb, idxb, rows, onesb, sem):
    c = lax.axis_index("c")
    s = lax.axis_index("s")
    w = s * NC + c

    zf = jnp.zeros((16,), jnp.float32)
    of = jnp.ones((16,), jnp.float32)
    lane = lax.broadcasted_iota(jnp.int32, (16,), 0)
    row0 = s * ROWS_PER_SUB

    def _fill_rows(val):
        def _body(k, _):
            rows[k // 8, pl.ds((k % 8) * 16, 16)] = val
            return 0
        lax.fori_loop(0, CHUNK * (D // 16), _body, 0)

    def _fill_idx():
        def _set(r):
            base = row0 + r * CHUNK
            for k in range(CHUNK // 16):
                idxb[0, pl.ds(k * 16, 16)] = base + k * 16 + lane
        return _set

    _set_idx = _fill_idx()

    def _zero_acc():
        _fill_rows(zf)
        for r in range(ROWS_PER_SUB // CHUNK):
            _set_idx(r)
            pltpu.sync_copy(rows, acc_sp.at[idxb.at[0]])

    def _writeback(out_hbm):
        for r in range(ROWS_PER_SUB // CHUNK):
            _set_idx(r)
            pltpu.async_copy(acc_sp.at[idxb.at[0]], rows, sem).wait()
            pltpu.sync_copy(
                rows, out_hbm.at[c, pl.ds(row0 + r * CHUNK, CHUNK)])

    _zero_acc()
    plsc.subcore_barrier()

    def _chunk_body(j, _):
        base = w * EW + j * CHUNK
        pltpu.sync_copy(src.at[pl.ds(base, CHUNK)], srcb.at[0])
        pltpu.sync_copy(dst.at[pl.ds(base, CHUNK)], dstb.at[0])
        pltpu.async_copy(t1.at[srcb.at[0]], rows, sem).wait()
        pltpu.sync_copy(rows, acc_sp.at[dstb.at[0]], add=True)
        return 0
    lax.fori_loop(0, NCHUNK, _chunk_body, 0)
    plsc.subcore_barrier()
    _writeback(acc_out)
    plsc.subcore_barrier()

    _zero_acc()

    def _fill_ones(k, _):
        onesb[k // 8, pl.ds((k % 8) * 16, 16)] = of
        return 0
    lax.fori_loop(0, CHUNK * (D // 16), _fill_ones, 0)
    plsc.subcore_barrier()

    def _deg_body(j, _):
        base = w * EW + j * CHUNK
        pltpu.sync_copy(dst.at[pl.ds(base, CHUNK)], dstb.at[0])
        pltpu.sync_copy(onesb, acc_sp.at[dstb.at[0]], add=True)
        return 0
    lax.fori_loop(0, NCHUNK, _deg_body, 0)
    plsc.subcore_barrier()
    _writeback(deg_out)


@functools.cache
def _build_sc_kernel():
    return pl.kernel(
        _segment_sum_sc_body,
        mesh=plsc.VectorSubcoreMesh(core_axis_name="c", subcore_axis_name="s"),
        out_type=[
            jax.ShapeDtypeStruct((NC, NP, D), jnp.float32),
            jax.ShapeDtypeStruct((NC, NP, D), jnp.float32),
        ],
        scratch_types=[
            pltpu.VMEM_SHARED((NP, D), jnp.float32),
            pltpu.VMEM((1, CHUNK), jnp.int32),
            pltpu.VMEM((1, CHUNK), jnp.int32),
            pltpu.VMEM((1, CHUNK), jnp.int32),
            pltpu.VMEM((CHUNK, D), jnp.float32),
            pltpu.VMEM((CHUNK, D), jnp.float32),
            pltpu.SemaphoreType.DMA,
        ],
    )


def kernel(x, edge_index, W1, b1, W2, b2):
    src = edge_index[0]
    dst = edge_index[1]
    t1 = pl.pallas_call(
        _tc1_body,
        out_shape=jax.ShapeDtypeStruct((N, D), jnp.float32),
    )(x, W1, b1.reshape(1, -1))
    acc, deg = _build_sc_kernel()(t1, src, dst)
    out = pl.pallas_call(
        _tc2_body,
        out_shape=jax.ShapeDtypeStruct((N, D), jnp.float32),
    )(acc, deg, W2, b2.reshape(1, -1))
    return out


# speedup vs baseline: 5.3608x; 1.1147x over previous
import functools

import jax
import jax.numpy as jnp
from jax import lax
from jax.experimental import pallas as pl
from jax.experimental.pallas import tpu as pltpu
from jax.experimental.pallas import tpu_sc as plsc

EPS = 1e-7
ARTANH_LIM = 1.0 - 1e-7

N, D = 10000, 128
E = 320000
NC, NS = 2, 16
NW = NC * NS
EW = E // NW
CHUNK = 80
NCHUNK = EW // CHUNK
NP = 10240
ROWS_PER_SUB = NP // NS


def _tc1_body(x_ref, w1_ref, b1_ref, out_ref):
    xb = x_ref[...]
    nrm = jnp.maximum(jnp.sqrt(jnp.sum(xb * xb, axis=1, keepdims=True)), EPS)
    z = jnp.minimum(nrm, ARTANH_LIM)
    scale = 0.5 * jnp.log((1.0 + z) / (1.0 - z)) / nrm
    t = xb * scale
    out_ref[...] = lax.dot_general(
        t, w1_ref[...], (((1,), (1,)), ((), ())),
        preferred_element_type=jnp.float32) + b1_ref[...]


def _tc2_body(acc_ref, deg_ref, w2_ref, b2_ref, out_ref):
    s = acc_ref[0, :N] + acc_ref[1, :N]
    deg = deg_ref[0, :N] + deg_ref[1, :N]
    agg = s / jnp.maximum(deg, 1.0)
    h = lax.dot_general(
        agg, w2_ref[...], (((1,), (1,)), ((), ())),
        preferred_element_type=jnp.float32) + b2_ref[...]
    nrm = jnp.maximum(jnp.sqrt(jnp.sum(h * h, axis=1, keepdims=True)), EPS)
    out_ref[...] = jnp.maximum(h * (jnp.tanh(nrm) / nrm), 0.0)


def _segment_sum_sc_body(t1, src, dst, acc_out, deg_out,
                         acc_sp, srcb, dstb, srcb2, dstb2, idxb,
                         rows, rows2, onesb, sem, sem2, sem3, sem4):
    c = lax.axis_index("c")
    s = lax.axis_index("s")
    w = s * NC + c

    zf = jnp.zeros((16,), jnp.float32)
    of = jnp.ones((16,), jnp.float32)
    lane = lax.broadcasted_iota(jnp.int32, (16,), 0)
    row0 = s * ROWS_PER_SUB

    def _fill_rows(val):
        def _body(k, _):
            rows[k // 8, pl.ds((k % 8) * 16, 16)] = val
            return 0
        lax.fori_loop(0, CHUNK * (D // 16), _body, 0)

    def _fill_idx():
        def _set(r):
            base = row0 + r * CHUNK
            for k in range(CHUNK // 16):
                idxb[0, pl.ds(k * 16, 16)] = base + k * 16 + lane
        return _set

    _set_idx = _fill_idx()

    def _zero_acc():
        _fill_rows(zf)
        for r in range(ROWS_PER_SUB // CHUNK):
            _set_idx(r)
            pltpu.sync_copy(rows, acc_sp.at[idxb.at[0]])

    def _writeback(out_hbm):
        for r in range(ROWS_PER_SUB // CHUNK):
            _set_idx(r)
            pltpu.async_copy(acc_sp.at[idxb.at[0]], rows, sem).wait()
            pltpu.sync_copy(
                rows, out_hbm.at[c, pl.ds(row0 + r * CHUNK, CHUNK)])

    _zero_acc()
    plsc.subcore_barrier()

    def _pair_body(i, _):
        b0 = w * EW + i * (2 * CHUNK)
        b1 = b0 + CHUNK
        pltpu.sync_copy(src.at[pl.ds(b0, CHUNK)], srcb.at[0])
        pltpu.sync_copy(dst.at[pl.ds(b0, CHUNK)], dstb.at[0])
        pltpu.sync_copy(src.at[pl.ds(b1, CHUNK)], srcb2.at[0])
        pltpu.sync_copy(dst.at[pl.ds(b1, CHUNK)], dstb2.at[0])
        ga = pltpu.async_copy(t1.at[srcb.at[0]], rows, sem)
        gb = pltpu.async_copy(t1.at[srcb2.at[0]], rows2, sem2)
        ga.wait()
        sa = pltpu.async_copy(rows, acc_sp.at[dstb.at[0]], sem3, add=True)
        gb.wait()
        sb = pltpu.async_copy(rows2, acc_sp.at[dstb2.at[0]], sem4, add=True)
        sa.wait()
        sb.wait()
        return 0
    lax.fori_loop(0, NCHUNK // 2, _pair_body, 0)
    tb = w * EW + (NCHUNK - 1) * CHUNK
    pltpu.sync_copy(src.at[pl.ds(tb, CHUNK)], srcb.at[0])
    pltpu.sync_copy(dst.at[pl.ds(tb, CHUNK)], dstb.at[0])
    pltpu.async_copy(t1.at[srcb.at[0]], rows, sem).wait()
    pltpu.sync_copy(rows, acc_sp.at[dstb.at[0]], add=True)
    plsc.subcore_barrier()
    _writeback(acc_out)
    plsc.subcore_barrier()

    _zero_acc()

    def _fill_ones(k, _):
        onesb[k // 8, pl.ds((k % 8) * 16, 16)] = of
        return 0
    lax.fori_loop(0, CHUNK * (D // 16), _fill_ones, 0)
    plsc.subcore_barrier()

    def _deg_pair(i, _):
        b0 = w * EW + i * (2 * CHUNK)
        b1 = b0 + CHUNK
        pltpu.sync_copy(dst.at[pl.ds(b0, CHUNK)], dstb.at[0])
        pltpu.sync_copy(dst.at[pl.ds(b1, CHUNK)], dstb2.at[0])
        sa = pltpu.async_copy(onesb, acc_sp.at[dstb.at[0]], sem3, add=True)
        sb = pltpu.async_copy(onesb, acc_sp.at[dstb2.at[0]], sem4, add=True)
        sa.wait()
        sb.wait()
        return 0
    lax.fori_loop(0, NCHUNK // 2, _deg_pair, 0)
    pltpu.sync_copy(dst.at[pl.ds(tb, CHUNK)], dstb.at[0])
    pltpu.sync_copy(onesb, acc_sp.at[dstb.at[0]], add=True)
    plsc.subcore_barrier()
    _writeback(deg_out)


@functools.cache
def _build_sc_kernel():
    return pl.kernel(
        _segment_sum_sc_body,
        mesh=plsc.VectorSubcoreMesh(core_axis_name="c", subcore_axis_name="s"),
        out_type=[
            jax.ShapeDtypeStruct((NC, NP, D), jnp.float32),
            jax.ShapeDtypeStruct((NC, NP, D), jnp.float32),
        ],
        scratch_types=[
            pltpu.VMEM_SHARED((NP, D), jnp.float32),
            pltpu.VMEM((1, CHUNK), jnp.int32),
            pltpu.VMEM((1, CHUNK), jnp.int32),
            pltpu.VMEM((1, CHUNK), jnp.int32),
            pltpu.VMEM((1, CHUNK), jnp.int32),
            pltpu.VMEM((1, CHUNK), jnp.int32),
            pltpu.VMEM((CHUNK, D), jnp.float32),
            pltpu.VMEM((CHUNK, D), jnp.float32),
            pltpu.VMEM((CHUNK, D), jnp.float32),
            pltpu.SemaphoreType.DMA,
            pltpu.SemaphoreType.DMA,
            pltpu.SemaphoreType.DMA,
            pltpu.SemaphoreType.DMA,
        ],
    )


def kernel(x, edge_index, W1, b1, W2, b2):
    src = edge_index[0]
    dst = edge_index[1]
    t1 = pl.pallas_call(
        _tc1_body,
        out_shape=jax.ShapeDtypeStruct((N, D), jnp.float32),
    )(x, W1, b1.reshape(1, -1))
    acc, deg = _build_sc_kernel()(t1, src, dst)
    out = pl.pallas_call(
        _tc2_body,
        out_shape=jax.ShapeDtypeStruct((N, D), jnp.float32),
    )(acc, deg, W2, b2.reshape(1, -1))
    return out
